# Initial kernel scaffold; baseline (speedup 1.0000x reference)
import jax
import jax.numpy as jnp
from jax.experimental import pallas as pl


def kernel(V, E, X, W, b):
    raise NotImplementedError("write your pallas kernel here")


# speedup vs baseline: 7.8450x; 7.8450x over previous
import functools

import jax
import jax.numpy as jnp
from jax import lax
from jax.experimental import pallas as pl
from jax.experimental.pallas import tpu as pltpu
from jax.experimental.pallas import tpu_sc as plsc

N_NODES = 10000
N_PAD = 10240
D = 128
NC = 2
NS = 16
NW = NC * NS
CHUNK = 80
N_EDGES = 320000
EDGES_PER_TILE = N_EDGES // NW
NCHUNKS = EDGES_PER_TILE // CHUNK
ROWS_PER_TILE = N_PAD // NS


def _sc_aggregate(E_r, X, Z):
    mesh = plsc.VectorSubcoreMesh(
        core_axis_name="c", subcore_axis_name="s", num_cores=NC, num_subcores=NS
    )

    @functools.partial(
        pl.kernel,
        mesh=mesh,
        out_type=jax.ShapeDtypeStruct((NC, N_PAD, D), jnp.float32),
        scratch_types=[
            pltpu.VMEM((NCHUNKS, CHUNK), jnp.int32),
            pltpu.VMEM((NCHUNKS, CHUNK), jnp.int32),
            pltpu.VMEM((CHUNK, D), jnp.float32),
            pltpu.VMEM_SHARED((N_PAD, D), jnp.float32),
            pltpu.SemaphoreType.DMA,
        ],
    )
    def k(e_hbm, x_hbm, z_hbm, out_hbm, src_v, dst_v, rows_v, agg_s, sem):
        c = lax.axis_index("c")
        s = lax.axis_index("s")
        wid = c * NS + s
        pltpu.sync_copy(e_hbm.at[0, wid], src_v)
        pltpu.sync_copy(e_hbm.at[1, wid], dst_v)
        r0 = s * ROWS_PER_TILE
        pltpu.sync_copy(
            z_hbm.at[pl.ds(r0, ROWS_PER_TILE)],
            agg_s.at[pl.ds(r0, ROWS_PER_TILE)],
        )
        plsc.subcore_barrier()

        def body(j, carry):
            pltpu.async_copy(x_hbm.at[src_v.at[j]], rows_v, sem).wait()
            pltpu.sync_copy(rows_v, agg_s.at[dst_v.at[j]], add=True)
            return carry

        lax.fori_loop(0, NCHUNKS, body, 0)
        plsc.subcore_barrier()
        pltpu.sync_copy(
            agg_s.at[pl.ds(r0, ROWS_PER_TILE)],
            out_hbm.at[c, pl.ds(r0, ROWS_PER_TILE)],
        )

    return k(E_r, X, Z)


def _tc_finish(P, W, b2):
    BLK = 1280
    grid = (N_PAD // BLK,)

    def body(p_ref, w_ref, b_ref, o_ref):
        a = p_ref[0] + p_ref[1]
        acc = jnp.dot(a, w_ref[...], preferred_element_type=jnp.float32)
        o_ref[...] = jnp.maximum(acc + b_ref[...], 0.0)

    return pl.pallas_call(
        body,
        grid=grid,
        in_specs=[
            pl.BlockSpec((2, BLK, D), lambda i: (0, i, 0)),
            pl.BlockSpec((D, D), lambda i: (0, 0)),
            pl.BlockSpec((1, D), lambda i: (0, 0)),
        ],
        out_specs=pl.BlockSpec((BLK, D), lambda i: (i, 0)),
        out_shape=jax.ShapeDtypeStruct((N_PAD, D), jnp.float32),
    )(P, W, b2)


def kernel(V, E, X, W, b):
    E_r = E.reshape(2, NW, NCHUNKS, CHUNK)
    Z = jnp.zeros((N_PAD, D), jnp.float32)
    P = _sc_aggregate(E_r, X, Z)
    out = _tc_finish(P, W, b.reshape(1, D))
    return out[:N_NODES]


# speedup vs baseline: 9.4035x; 1.1987x over previous
import functools

import jax
import jax.numpy as jnp
from jax import lax
from jax.experimental import pallas as pl
from jax.experimental.pallas import tpu as pltpu
from jax.experimental.pallas import tpu_sc as plsc

N_NODES = 10000
N_PAD = 10240
D = 128
NC = 2
NS = 16
NW = NC * NS
CHUNK = 80
N_EDGES = 320000
EDGES_PER_TILE = N_EDGES // NW
NCHUNKS = EDGES_PER_TILE // CHUNK
ROWS_PER_TILE = N_PAD // NS


def _sc_aggregate(E_src, E_dst, X, Z):
    mesh = plsc.VectorSubcoreMesh(
        core_axis_name="c", subcore_axis_name="s", num_cores=NC, num_subcores=NS
    )

    @functools.partial(
        pl.kernel,
        mesh=mesh,
        out_type=jax.ShapeDtypeStruct((NC, N_PAD, D), jnp.float32),
        scratch_types=[
            pltpu.VMEM((EDGES_PER_TILE,), jnp.int32),
            pltpu.VMEM((NCHUNKS, CHUNK), jnp.int32),
            pltpu.VMEM((2, CHUNK, D), jnp.float32),
            pltpu.VMEM_SHARED((N_PAD, D), jnp.float32),
            pltpu.SemaphoreType.DMA,
            pltpu.SemaphoreType.DMA,
        ],
    )
    def k(es_hbm, ed_hbm, x_hbm, z_hbm, out_hbm, src_v, dst_v, rows_v,
          agg_s, sem_a, sem_b):
        rows_a = rows_v.at[0]
        rows_b = rows_v.at[1]
        c = lax.axis_index("c")
        s = lax.axis_index("s")
        wid = c * NS + s
        pltpu.sync_copy(es_hbm.at[wid], src_v)
        pltpu.sync_copy(ed_hbm.at[wid], dst_v)
        r0 = s * ROWS_PER_TILE
        pltpu.sync_copy(
            z_hbm.at[pl.ds(r0, ROWS_PER_TILE)],
            agg_s.at[pl.ds(r0, ROWS_PER_TILE)],
        )
        plsc.subcore_barrier()

        def sidx(j):
            return src_v.at[pl.ds(j * CHUNK, CHUNK)]

        pltpu.async_copy(x_hbm.at[sidx(0)], rows_a, sem_a)

        def body(i, carry):
            j = 2 * i
            pltpu.make_async_copy(x_hbm.at[sidx(j)], rows_a, sem_a).wait()
            pltpu.async_copy(x_hbm.at[sidx(j + 1)], rows_b, sem_b)
            pltpu.sync_copy(rows_a, agg_s.at[dst_v.at[j]], add=True)
            pltpu.make_async_copy(x_hbm.at[sidx(j + 1)], rows_b, sem_b).wait()
            pltpu.async_copy(x_hbm.at[sidx(j + 2)], rows_a, sem_a)
            pltpu.sync_copy(rows_b, agg_s.at[dst_v.at[j + 1]], add=True)
            return carry

        lax.fori_loop(0, NCHUNKS // 2, body, 0)
        pltpu.make_async_copy(
            x_hbm.at[sidx(NCHUNKS - 1)], rows_a, sem_a).wait()
        pltpu.sync_copy(rows_a, agg_s.at[dst_v.at[NCHUNKS - 1]], add=True)
        plsc.subcore_barrier()
        pltpu.sync_copy(
            agg_s.at[pl.ds(r0, ROWS_PER_TILE)],
            out_hbm.at[c, pl.ds(r0, ROWS_PER_TILE)],
        )

    return k(E_src, E_dst, X, Z)


def _tc_finish(P, W, b2):
    BLK = 1280
    grid = (N_PAD // BLK,)

    def body(p_ref, w_ref, b_ref, o_ref):
        a = p_ref[0] + p_ref[1]
        acc = jnp.dot(a, w_ref[...], preferred_element_type=jnp.float32)
        o_ref[...] = jnp.maximum(acc + b_ref[...], 0.0)

    return pl.pallas_call(
        body,
        grid=grid,
        in_specs=[
            pl.BlockSpec((2, BLK, D), lambda i: (0, i, 0)),
            pl.BlockSpec((D, D), lambda i: (0, 0)),
            pl.BlockSpec((1, D), lambda i: (0, 0)),
        ],
        out_specs=pl.BlockSpec((BLK, D), lambda i: (i, 0)),
        out_shape=jax.ShapeDtypeStruct((N_PAD, D), jnp.float32),
    )(P, W, b2)


def kernel(V, E, X, W, b):
    E_src = E[0].reshape(NW, EDGES_PER_TILE)
    E_dst = E[1].reshape(NW, NCHUNKS, CHUNK)
    Z = jnp.zeros((N_PAD, D), jnp.float32)
    P = _sc_aggregate(E_src, E_dst, X, Z)
    out = _tc_finish(P, W, b.reshape(1, D))
    return out[:N_NODES]
